# baseline (speedup 1.0000x reference)
import functools

import jax
import jax.numpy as jnp
from jax import lax
from jax.experimental import pallas as pl
from jax.experimental.pallas import tpu as pltpu
from jax.experimental.pallas import tpu_sc as plsc

_LANES = 16
_NUM_CORES = 2
_NUM_SUBCORES = 16
_NW = _NUM_CORES * _NUM_SUBCORES


@functools.lru_cache(maxsize=None)
def _build(B, S, D, G):
    rows_per_w = B // _NW
    n_chunks = rows_per_w // G
    CH = G * S
    mesh = plsc.VectorSubcoreMesh(core_axis_name="c", subcore_axis_name="s")

    @functools.partial(
        pl.kernel,
        out_type=jax.ShapeDtypeStruct((B * S, D), jnp.float32),
        mesh=mesh,
        compiler_params=pltpu.CompilerParams(use_tc_tiling_on_sc=False),
        scratch_types=[
            pltpu.VMEM((CH,), jnp.int32),
            pltpu.VMEM((CH, D), jnp.float32),
            pltpu.VMEM((S, D), jnp.float32),
            pltpu.SemaphoreType.DMA,
        ],
    )
    def emb_kernel(x_hbm, pos_hbm, table_hbm, out_hbm, idx_v, rows_v, pos_v, sem):
        wid = lax.axis_index("s") * _NUM_CORES + lax.axis_index("c")
        base = wid * rows_per_w * S
        pltpu.sync_copy(pos_hbm, pos_v)
        scale = jnp.full((_LANES,), 8.0, dtype=jnp.float32)

        def chunk_body(ci, _):
            off = base + ci * CH
            pltpu.sync_copy(x_hbm.at[pl.ds(off, CH)], idx_v)
            pltpu.async_copy(table_hbm.at[idx_v], rows_v, sem).wait()

            def pos_body(s, _):
                for g in range(G):
                    row = g * S + s
                    for c in range(D // _LANES):
                        sl = pl.ds(c * _LANES, _LANES)
                        rows_v[row, sl] = rows_v[row, sl] * scale + pos_v[s, sl]
                return 0

            lax.fori_loop(0, S, pos_body, 0, unroll=False)
            pltpu.sync_copy(rows_v, out_hbm.at[pl.ds(off, CH)])
            return 0

        lax.fori_loop(0, n_chunks, chunk_body, 0, unroll=False)

    return emb_kernel


def kernel(x, table, pos_enc, training):
    B, S = x.shape
    V, D = table.shape
    pos = pos_enc[0, :S, :]
    x_flat = x.reshape(B * S)
    out = _build(B, S, D, 4)(x_flat, pos, table)
    return out.reshape(B, S, D)
